# Initial kernel scaffold; baseline (speedup 1.0000x reference)
import jax
import jax.numpy as jnp
from jax.experimental import pallas as pl


def kernel(x, edge_index, W1, a_src1, a_dst1, b1, W2, a_src2, a_dst2, b2):
    raise NotImplementedError("write your pallas kernel here")


# speedup vs baseline: 4.8558x; 4.8558x over previous
import functools

import jax
import jax.numpy as jnp
from jax import lax
from jax.experimental import pallas as pl
from jax.experimental.pallas import tpu as pltpu

N = 10000
E = 320000
IN_CH = 128
HID = 32
HEADS = 8
OUT_CH = 128

BLK = 400
ACC_W = 144


def _lrelu_exp(s):
    return jnp.exp(jnp.maximum(s, 0.2 * s))


def _tc1_body(x_ref, w1_ref, m1a_ref, m1b_ref, h_ref, a_ref, b_ref):
    h = jnp.dot(x_ref[...], w1_ref[...], preferred_element_type=jnp.float32)
    h_ref[...] = h
    a_ref[...] = jnp.dot(h, m1a_ref[...], preferred_element_type=jnp.float32)
    b_ref[...] = jnp.dot(h, m1b_ref[...], preferred_element_type=jnp.float32)


def _tc1(x, W1, M1a, M1b):
    return pl.pallas_call(
        _tc1_body,
        grid=(N // BLK,),
        in_specs=[
            pl.BlockSpec((BLK, IN_CH), lambda i: (i, 0)),
            pl.BlockSpec((IN_CH, HEADS * HID), lambda i: (0, 0)),
            pl.BlockSpec((HEADS * HID, 16), lambda i: (0, 0)),
            pl.BlockSpec((HEADS * HID, 16), lambda i: (0, 0)),
        ],
        out_specs=[
            pl.BlockSpec((BLK, HEADS * HID), lambda i: (i, 0)),
            pl.BlockSpec((BLK, 16), lambda i: (i, 0)),
            pl.BlockSpec((BLK, 16), lambda i: (i, 0)),
        ],
        out_shape=[
            jax.ShapeDtypeStruct((N, HEADS * HID), jnp.float32),
            jax.ShapeDtypeStruct((N, 16), jnp.float32),
            jax.ShapeDtypeStruct((N, 16), jnp.float32),
        ],
    )(x, W1, M1a, M1b)


def _tc2_body(accA_ref, accB_ref, asdA_ref, h1_ref, b1_ref, w2_ref,
              e8_ref, p2a_ref, p2b_ref, h2_ref, a2_ref, b2t_ref):
    asd = asdA_ref[...]
    ws = _lrelu_exp(asd[:, :8] + asd[:, 8:16])
    accA = accA_ref[...]
    accB = accB_ref[...]
    numer = jnp.concatenate([accA[:, :128], accB[:, :128]], axis=1)
    den8 = accA[:, 128:136] + ws
    e8 = e8_ref[...]
    numer = numer + jnp.dot(ws, e8, preferred_element_type=jnp.float32) * h1_ref[...]
    dene = jnp.dot(den8, e8, preferred_element_type=jnp.float32)
    o1 = numer / dene + b1_ref[...]
    g = jnp.where(o1 > 0, o1, jnp.exp(jnp.minimum(o1, 0.0)) - 1.0)
    h2 = jnp.dot(g, w2_ref[...], preferred_element_type=jnp.float32)
    h2_ref[...] = h2
    a2_ref[...] = jnp.dot(h2, p2a_ref[...], preferred_element_type=jnp.float32)
    b2t_ref[...] = jnp.dot(h2, p2b_ref[...], preferred_element_type=jnp.float32)


def _tc2(accA, accB, asdA, h1, b1, W2, E8, P2a, P2b):
    return pl.pallas_call(
        _tc2_body,
        grid=(N // BLK,),
        in_specs=[
            pl.BlockSpec((BLK, ACC_W), lambda i: (i, 0)),
            pl.BlockSpec((BLK, ACC_W), lambda i: (i, 0)),
            pl.BlockSpec((BLK, 16), lambda i: (i, 0)),
            pl.BlockSpec((BLK, HEADS * HID), lambda i: (i, 0)),
            pl.BlockSpec((1, HEADS * HID), lambda i: (0, 0)),
            pl.BlockSpec((HEADS * HID, OUT_CH), lambda i: (0, 0)),
            pl.BlockSpec((HEADS, HEADS * HID), lambda i: (0, 0)),
            pl.BlockSpec((OUT_CH, 16), lambda i: (0, 0)),
            pl.BlockSpec((OUT_CH, 16), lambda i: (0, 0)),
        ],
        out_specs=[
            pl.BlockSpec((BLK, OUT_CH), lambda i: (i, 0)),
            pl.BlockSpec((BLK, 16), lambda i: (i, 0)),
            pl.BlockSpec((BLK, 16), lambda i: (i, 0)),
        ],
        out_shape=[
            jax.ShapeDtypeStruct((N, OUT_CH), jnp.float32),
            jax.ShapeDtypeStruct((N, 16), jnp.float32),
            jax.ShapeDtypeStruct((N, 16), jnp.float32),
        ],
    )(accA, accB, asdA, h1, b1, W2, E8, P2a, P2b)


def _tc3_body(accA_ref, accB_ref, asd2A_ref, asd2B_ref, h2_ref, b2_ref, out_ref):
    s2 = asd2A_ref[...][:, 0:1] + asd2B_ref[...][:, 0:1]
    ws2 = _lrelu_exp(s2)
    accA = accA_ref[...]
    accB = accB_ref[...]
    numer = accA[:, :128] + accB[:, :128] + ws2 * h2_ref[...]
    den = accA[:, 128:129] + accB[:, 128:129] + ws2
    out_ref[...] = numer / den + b2_ref[...]


def _tc3(accA, accB, asd2A, asd2B, h2, b2):
    return pl.pallas_call(
        _tc3_body,
        grid=(N // BLK,),
        in_specs=[
            pl.BlockSpec((BLK, ACC_W), lambda i: (i, 0)),
            pl.BlockSpec((BLK, ACC_W), lambda i: (i, 0)),
            pl.BlockSpec((BLK, 16), lambda i: (i, 0)),
            pl.BlockSpec((BLK, 16), lambda i: (i, 0)),
            pl.BlockSpec((BLK, OUT_CH), lambda i: (i, 0)),
            pl.BlockSpec((1, OUT_CH), lambda i: (0, 0)),
        ],
        out_specs=pl.BlockSpec((BLK, OUT_CH), lambda i: (i, 0)),
        out_shape=jax.ShapeDtypeStruct((N, OUT_CH), jnp.float32),
    )(accA, accB, asd2A, asd2B, h2, b2)


def kernel(x, edge_index, W1, a_src1, a_dst1, b1, W2, a_src2, a_dst2, b2):
    src = edge_index[0]
    dst = edge_index[1]

    head_of = jnp.arange(HEADS * HID) // HID
    oh = (head_of[:, None] == jnp.arange(HEADS)[None, :]).astype(jnp.float32)
    A1s = a_src1.reshape(-1)[:, None] * oh
    A1d = a_dst1.reshape(-1)[:, None] * oh
    z8 = jnp.zeros((HEADS * HID, 8), jnp.float32)
    M1a = jnp.concatenate([A1s, A1d], axis=1)
    M1b = jnp.concatenate([A1d, z8], axis=1)

    E8 = jnp.repeat(jnp.eye(HEADS, dtype=jnp.float32), HID, axis=1)
    P2a = jnp.concatenate([a_src2.reshape(OUT_CH, 1),
                           jnp.zeros((OUT_CH, 15), jnp.float32)], axis=1)
    P2b = jnp.concatenate([a_dst2.reshape(OUT_CH, 1),
                           jnp.zeros((OUT_CH, 15), jnp.float32)], axis=1)

    h1, asdA, asdB = _tc1(x, W1, M1a, M1b)

    als1 = asdA[:, :8]
    ald1 = asdA[:, 8:16]
    s = als1[src] + ald1[dst]
    w = _lrelu_exp(s)
    den = jax.ops.segment_sum(w, dst, num_segments=N)
    wexp = jnp.repeat(w, HID, axis=1)
    numer = jax.ops.segment_sum(h1[src] * wexp, dst, num_segments=N)
    accA = jnp.concatenate(
        [numer[:, :128], den, jnp.zeros((N, 8), jnp.float32)], axis=1)
    accB = jnp.concatenate(
        [numer[:, 128:], den, jnp.zeros((N, 8), jnp.float32)], axis=1)

    h2, asd2A, asd2B = _tc2(accA, accB, asdA, h1, b1.reshape(1, -1), W2,
                            E8, P2a, P2b)

    s2 = asd2A[:, 0][src] + asd2B[:, 0][dst]
    w2 = _lrelu_exp(s2)
    den2 = jax.ops.segment_sum(w2, dst, num_segments=N)
    numer2 = jax.ops.segment_sum(h2[src] * w2[:, None], dst, num_segments=N)
    acc2A = jnp.concatenate(
        [numer2, den2[:, None], jnp.zeros((N, 15), jnp.float32)], axis=1)
    acc2B = jnp.zeros((N, ACC_W), jnp.float32)

    return _tc3(acc2A, acc2B, asd2A, asd2B, h2, b2.reshape(1, -1))


# speedup vs baseline: 21.2752x; 4.3814x over previous
import functools

import jax
import jax.numpy as jnp
from jax import lax
from jax.experimental import pallas as pl
from jax.experimental.pallas import tpu as pltpu
from jax.experimental.pallas import tpu_sc as plsc

N = 10000
E = 320000
IN_CH = 128
HID = 32
HEADS = 8
OUT_CH = 128

BLK = 400
ACC_W = 144


def _lrelu_exp(s):
    return jnp.exp(jnp.maximum(s, 0.2 * s))


def _tc1_body(x_ref, w1_ref, m1a_ref, m1b_ref, h_ref, a_ref, b_ref):
    h = jnp.dot(x_ref[...], w1_ref[...], preferred_element_type=jnp.float32)
    h_ref[...] = h
    a_ref[...] = jnp.dot(h, m1a_ref[...], preferred_element_type=jnp.float32)
    b_ref[...] = jnp.dot(h, m1b_ref[...], preferred_element_type=jnp.float32)


def _tc1(x, W1, M1a, M1b):
    return pl.pallas_call(
        _tc1_body,
        grid=(N // BLK,),
        in_specs=[
            pl.BlockSpec((BLK, IN_CH), lambda i: (i, 0)),
            pl.BlockSpec((IN_CH, HEADS * HID), lambda i: (0, 0)),
            pl.BlockSpec((HEADS * HID, 16), lambda i: (0, 0)),
            pl.BlockSpec((HEADS * HID, 16), lambda i: (0, 0)),
        ],
        out_specs=[
            pl.BlockSpec((BLK, HEADS * HID), lambda i: (i, 0)),
            pl.BlockSpec((BLK, 16), lambda i: (i, 0)),
            pl.BlockSpec((BLK, 16), lambda i: (i, 0)),
        ],
        out_shape=[
            jax.ShapeDtypeStruct((N, HEADS * HID), jnp.float32),
            jax.ShapeDtypeStruct((N, 16), jnp.float32),
            jax.ShapeDtypeStruct((N, 16), jnp.float32),
        ],
    )(x, W1, M1a, M1b)


def _tc2_body(accA_ref, accB_ref, asdA_ref, h1_ref, b1_ref, w2_ref,
              e8_ref, p2a_ref, p2b_ref, h2_ref, a2_ref, b2t_ref):
    asd = asdA_ref[...]
    ws = _lrelu_exp(asd[:, :8] + asd[:, 8:16])
    accA = accA_ref[...]
    accB = accB_ref[...]
    numer = jnp.concatenate([accA[:, :128], accB[:, :128]], axis=1)
    den8 = accA[:, 128:136] + ws
    e8 = e8_ref[...]
    numer = numer + jnp.dot(ws, e8, preferred_element_type=jnp.float32) * h1_ref[...]
    dene = jnp.dot(den8, e8, preferred_element_type=jnp.float32)
    o1 = numer / dene + b1_ref[...]
    g = jnp.where(o1 > 0, o1, jnp.exp(jnp.minimum(o1, 0.0)) - 1.0)
    h2 = jnp.dot(g, w2_ref[...], preferred_element_type=jnp.float32)
    h2_ref[...] = h2
    a2_ref[...] = jnp.dot(h2, p2a_ref[...], preferred_element_type=jnp.float32)
    b2t_ref[...] = jnp.dot(h2, p2b_ref[...], preferred_element_type=jnp.float32)


def _tc2(accA, accB, asdA, h1, b1, W2, E8, P2a, P2b):
    return pl.pallas_call(
        _tc2_body,
        grid=(N // BLK,),
        in_specs=[
            pl.BlockSpec((BLK, ACC_W), lambda i: (i, 0)),
            pl.BlockSpec((BLK, ACC_W), lambda i: (i, 0)),
            pl.BlockSpec((BLK, 16), lambda i: (i, 0)),
            pl.BlockSpec((BLK, HEADS * HID), lambda i: (i, 0)),
            pl.BlockSpec((1, HEADS * HID), lambda i: (0, 0)),
            pl.BlockSpec((HEADS * HID, OUT_CH), lambda i: (0, 0)),
            pl.BlockSpec((HEADS, HEADS * HID), lambda i: (0, 0)),
            pl.BlockSpec((OUT_CH, 16), lambda i: (0, 0)),
            pl.BlockSpec((OUT_CH, 16), lambda i: (0, 0)),
        ],
        out_specs=[
            pl.BlockSpec((BLK, OUT_CH), lambda i: (i, 0)),
            pl.BlockSpec((BLK, 16), lambda i: (i, 0)),
            pl.BlockSpec((BLK, 16), lambda i: (i, 0)),
        ],
        out_shape=[
            jax.ShapeDtypeStruct((N, OUT_CH), jnp.float32),
            jax.ShapeDtypeStruct((N, 16), jnp.float32),
            jax.ShapeDtypeStruct((N, 16), jnp.float32),
        ],
    )(accA, accB, asdA, h1, b1, W2, E8, P2a, P2b)


def _tc3_body(accA_ref, accB_ref, asd2A_ref, asd2B_ref, h2_ref, b2_ref, out_ref):
    s2 = asd2A_ref[...][:, 0:1] + asd2B_ref[...][:, 0:1]
    ws2 = _lrelu_exp(s2)
    accA = accA_ref[...]
    accB = accB_ref[...]
    numer = accA[:, :128] + accB[:, :128] + ws2 * h2_ref[...]
    den = accA[:, 128:129] + accB[:, 128:129] + ws2
    out_ref[...] = numer / den + b2_ref[...]


def _tc3(accA, accB, asd2A, asd2B, h2, b2):
    return pl.pallas_call(
        _tc3_body,
        grid=(N // BLK,),
        in_specs=[
            pl.BlockSpec((BLK, ACC_W), lambda i: (i, 0)),
            pl.BlockSpec((BLK, ACC_W), lambda i: (i, 0)),
            pl.BlockSpec((BLK, 16), lambda i: (i, 0)),
            pl.BlockSpec((BLK, 16), lambda i: (i, 0)),
            pl.BlockSpec((BLK, OUT_CH), lambda i: (i, 0)),
            pl.BlockSpec((1, OUT_CH), lambda i: (0, 0)),
        ],
        out_specs=pl.BlockSpec((BLK, OUT_CH), lambda i: (i, 0)),
        out_shape=jax.ShapeDtypeStruct((N, OUT_CH), jnp.float32),
    )(accA, accB, asd2A, asd2B, h2, b2)


K_E = 128
NCHUNK = E // K_E
NSUB = 16
NPAD = 10112
ROWS_T = NPAD // NSUB


def _dyn_gather16(x, idx):
    return lax.gather(
        x, idx[:, None],
        lax.GatherDimensionNumbers(offset_dims=(), collapsed_slice_dims=(0,),
                                   start_index_map=(0,)),
        (1,), mode=lax.GatherScatterMode.PROMISE_IN_BOUNDS)


def _sc_layer1(h1r, asdA, asdB, src, dst, zeros):
    mesh = plsc.VectorSubcoreMesh(core_axis_name="c", subcore_axis_name="s")

    @functools.partial(
        pl.kernel,
        out_type=jax.ShapeDtypeStruct((2, NPAD, ACC_W), jnp.float32),
        mesh=mesh,
        compiler_params=pltpu.CompilerParams(use_tc_tiling_on_sc=False),
        scratch_types=[
            pltpu.VMEM((K_E,), jnp.int32),
            pltpu.VMEM((K_E,), jnp.int32),
            pltpu.VMEM((K_E,), jnp.int32),
            pltpu.VMEM((K_E, 16), jnp.float32),
            pltpu.VMEM((K_E, 16), jnp.float32),
            pltpu.VMEM((K_E, 128), jnp.float32),
            pltpu.VMEM((K_E, ACC_W), jnp.float32),
            pltpu.VMEM_SHARED((NPAD, ACC_W), jnp.float32),
            pltpu.SemaphoreType.DMA,
            pltpu.SemaphoreType.DMA,
            pltpu.SemaphoreType.DMA,
        ],
    )
    def k(h1r_hbm, asdA_hbm, asdB_hbm, src_hbm, dst_hbm, zeros_hbm, out_hbm,
          idx_s, idx_d, idx2, abuf, bbuf, hbuf, sbuf, acc, sem1, sem2, sem3):
        c = lax.axis_index("c")
        s = lax.axis_index("s")
        r0 = s * ROWS_T
        pltpu.sync_copy(zeros_hbm.at[pl.ds(r0, ROWS_T)],
                        acc.at[pl.ds(r0, ROWS_T)])
        plsc.subcore_barrier()
        lane = lax.iota(jnp.int32, 16)
        msk8 = lane < 8

        def chunk_body(i, carry):
            cid = s + i * NSUB

            @pl.when(cid < NCHUNK)
            def _():
                off = cid * K_E
                pltpu.sync_copy(src_hbm.at[pl.ds(off, K_E)], idx_s)
                pltpu.sync_copy(dst_hbm.at[pl.ds(off, K_E)], idx_d)
                for j in range(K_E // 16):
                    idx2[pl.ds(16 * j, 16)] = idx_s[pl.ds(16 * j, 16)] * 2 + c
                cp1 = pltpu.async_copy(asdA_hbm.at[idx_s], abuf, sem1)
                cp2 = pltpu.async_copy(asdB_hbm.at[idx_d], bbuf, sem2)
                cp3 = pltpu.async_copy(h1r_hbm.at[idx2], hbuf, sem3)
                cp1.wait()
                cp2.wait()
                cp3.wait()

                def edge(kk, carry2):
                    sv = abuf[kk] + bbuf[kk]
                    w = jnp.exp(jnp.maximum(sv, 0.2 * sv))
                    sbuf[kk, pl.ds(128, 16)] = jnp.where(msk8, w, 0.0)
                    for m in range(4):
                        sp = _dyn_gather16(
                            w, jnp.full((16,), 4 * c + m, jnp.int32))
                        sbuf[kk, pl.ds(32 * m, 16)] = (
                            hbuf[kk, pl.ds(32 * m, 16)] * sp)
                        sbuf[kk, pl.ds(32 * m + 16, 16)] = (
                            hbuf[kk, pl.ds(32 * m + 16, 16)] * sp)
                    return carry2

                lax.fori_loop(0, K_E, edge, 0)
                pltpu.sync_copy(sbuf, acc.at[idx_d], add=True)

            return carry

        lax.fori_loop(0, (NCHUNK + NSUB - 1) // NSUB, chunk_body, 0)
        plsc.subcore_barrier()
        pltpu.sync_copy(acc.at[pl.ds(r0, ROWS_T)],
                        out_hbm.at[c, pl.ds(r0, ROWS_T)])

    return k(h1r, asdA, asdB, src, dst, zeros)


def _sc_layer2(h2, asd2A, asd2B, src, dst, zeros):
    mesh = plsc.VectorSubcoreMesh(core_axis_name="c", subcore_axis_name="s")

    @functools.partial(
        pl.kernel,
        out_type=jax.ShapeDtypeStruct((2, NPAD, ACC_W), jnp.float32),
        mesh=mesh,
        compiler_params=pltpu.CompilerParams(use_tc_tiling_on_sc=False),
        scratch_types=[
            pltpu.VMEM((K_E,), jnp.int32),
            pltpu.VMEM((K_E,), jnp.int32),
            pltpu.VMEM((K_E, 16), jnp.float32),
            pltpu.VMEM((K_E, 16), jnp.float32),
            pltpu.VMEM((K_E, 128), jnp.float32),
            pltpu.VMEM((K_E, ACC_W), jnp.float32),
            pltpu.VMEM_SHARED((NPAD, ACC_W), jnp.float32),
            pltpu.SemaphoreType.DMA,
            pltpu.SemaphoreType.DMA,
            pltpu.SemaphoreType.DMA,
        ],
    )
    def k(h2_hbm, asdA_hbm, asdB_hbm, src_hbm, dst_hbm, zeros_hbm, out_hbm,
          idx_s, idx_d, abuf, bbuf, hbuf, sbuf, acc, sem1, sem2, sem3):
        c = lax.axis_index("c")
        s = lax.axis_index("s")
        wid = s * 2 + c
        r0 = s * ROWS_T
        pltpu.sync_copy(zeros_hbm.at[pl.ds(r0, ROWS_T)],
                        acc.at[pl.ds(r0, ROWS_T)])
        plsc.subcore_barrier()
        lane = lax.iota(jnp.int32, 16)
        msk0 = lane < 1
        sp_idx = jnp.zeros((16,), jnp.int32)

        def chunk_body(i, carry):
            cid = wid + i * 32

            @pl.when(cid < NCHUNK)
            def _():
                off = cid * K_E
                pltpu.sync_copy(src_hbm.at[pl.ds(off, K_E)], idx_s)
                pltpu.sync_copy(dst_hbm.at[pl.ds(off, K_E)], idx_d)
                cp1 = pltpu.async_copy(asdA_hbm.at[idx_s], abuf, sem1)
                cp2 = pltpu.async_copy(asdB_hbm.at[idx_d], bbuf, sem2)
                cp3 = pltpu.async_copy(h2_hbm.at[idx_s], hbuf, sem3)
                cp1.wait()
                cp2.wait()
                cp3.wait()

                def edge(kk, carry2):
                    sv = abuf[kk] + bbuf[kk]
                    w = jnp.exp(jnp.maximum(sv, 0.2 * sv))
                    sbuf[kk, pl.ds(128, 16)] = jnp.where(msk0, w, 0.0)
                    sp = _dyn_gather16(w, sp_idx)
                    for j in range(8):
                        sbuf[kk, pl.ds(16 * j, 16)] = (
                            hbuf[kk, pl.ds(16 * j, 16)] * sp)
                    return carry2

                lax.fori_loop(0, K_E, edge, 0)
                pltpu.sync_copy(sbuf, acc.at[idx_d], add=True)

            return carry

        lax.fori_loop(0, (NCHUNK + 31) // 32, chunk_body, 0)
        plsc.subcore_barrier()
        pltpu.sync_copy(acc.at[pl.ds(r0, ROWS_T)],
                        out_hbm.at[c, pl.ds(r0, ROWS_T)])

    return k(h2, asd2A, asd2B, src, dst, zeros)


def kernel(x, edge_index, W1, a_src1, a_dst1, b1, W2, a_src2, a_dst2, b2):
    src = edge_index[0]
    dst = edge_index[1]

    head_of = jnp.arange(HEADS * HID) // HID
    oh = (head_of[:, None] == jnp.arange(HEADS)[None, :]).astype(jnp.float32)
    A1s = a_src1.reshape(-1)[:, None] * oh
    A1d = a_dst1.reshape(-1)[:, None] * oh
    z8 = jnp.zeros((HEADS * HID, 8), jnp.float32)
    M1a = jnp.concatenate([A1s, A1d], axis=1)
    M1b = jnp.concatenate([A1d, z8], axis=1)

    E8 = jnp.repeat(jnp.eye(HEADS, dtype=jnp.float32), HID, axis=1)
    P2a = jnp.concatenate([a_src2.reshape(OUT_CH, 1),
                           jnp.zeros((OUT_CH, 15), jnp.float32)], axis=1)
    P2b = jnp.concatenate([a_dst2.reshape(OUT_CH, 1),
                           jnp.zeros((OUT_CH, 15), jnp.float32)], axis=1)

    h1, asdA, asdB = _tc1(x, W1, M1a, M1b)

    zeros = jnp.zeros((NPAD, ACC_W), jnp.float32)

    h1r = h1.reshape(2 * N, 128)
    acc1 = _sc_layer1(h1r, asdA, asdB, src, dst, zeros)

    h2, asd2A, asd2B = _tc2(acc1[0], acc1[1], asdA, h1, b1.reshape(1, -1), W2,
                            E8, P2a, P2b)

    acc2 = _sc_layer2(h2, asd2A, asd2B, src, dst, zeros)

    return _tc3(acc2[0], acc2[1], asd2A, asd2B, h2, b2.reshape(1, -1))
